# Initial kernel scaffold; baseline (speedup 1.0000x reference)
import jax
import jax.numpy as jnp
from jax.experimental import pallas as pl


def kernel(w_sigma, w_rgb, rays):
    raise NotImplementedError("write your pallas kernel here")


# speedup vs baseline: 96.1996x; 96.1996x over previous
import functools

import numpy as np
import jax
import jax.numpy as jnp
from jax import lax
from jax.experimental import pallas as pl
from jax.experimental.pallas import tpu as pltpu
from jax.experimental.pallas import tpu_sc as plsc

_GRID = 128
_S = 256
_N = 8192
_PI = np.pi
_BIG = 1e30

_NC = 2
_NS = 16
_NW = _NC * _NS
_TOT = _S * _N
_PERW = _TOT // _NW
_K = 8192
_NCH = _PERW // _K

_R1 = 512
_R2 = 512


def _ray_setup(rays_ref):
    ox = rays_ref[0:1, :]
    oy = rays_ref[1:2, :]
    oz = rays_ref[2:3, :]
    rx = rays_ref[3:4, :]
    ry = rays_ref[4:5, :]
    rz = rays_ref[5:6, :]
    norm = jnp.sqrt(rx * rx + ry * ry + rz * rz)
    nears = []
    fars = []
    oks = []
    ds = []
    os_ = (ox, oy, oz)
    for o, draw in zip(os_, (rx, ry, rz)):
        d = draw / norm
        ds.append(d)
        zero = d == 0.0
        safe_d = jnp.where(zero, 1.0, d)
        i1 = jnp.where(zero, -_BIG, (-1.5 - o) / safe_d)
        i2 = jnp.where(zero, _BIG, (1.5 - o) / safe_d)
        nears.append(jnp.minimum(i1, i2))
        fars.append(jnp.maximum(i1, i2))
        inside = (o >= -1.5) & (o <= 1.5)
        oks.append(jnp.logical_or(~zero, inside))
    near = jnp.maximum(jnp.maximum(nears[0], nears[1]), nears[2])
    far = jnp.minimum(jnp.minimum(fars[0], fars[1]), fars[2])
    ok = oks[0] & oks[1] & oks[2]
    isect = (near <= far) & ok
    ns = jnp.where(isect,
                   jnp.minimum((far - near) * 32.0, 256.0).astype(jnp.int32),
                   0)
    ns_f = jnp.maximum(ns, 1).astype(jnp.float32)
    return os_, ds, near, far, ns, ns_f


def _idx_body(rays_ref, idx_ref):
    os_, ds, near, far, ns, ns_f = _ray_setup(rays_ref)
    j = lax.broadcasted_iota(jnp.int32, (_S, _R1), 0).astype(jnp.float32)
    t = near + (far - near) * (j + 0.5) / ns_f
    ips = []
    for o, d in zip(os_, ds):
        pos = (o + d * t) / 1.5 * 0.5 + 0.5
        p = pos * float(_GRID)
        ip = jnp.clip(jnp.floor(p).astype(jnp.int32), 0, _GRID - 1)
        ips.append(ip)
    idx_ref[...] = (ips[0] * _GRID + ips[1]) * _GRID + ips[2]


@jax.jit
def _idx_call(rays_t):
    return pl.pallas_call(
        _idx_body,
        grid=(_N // _R1,),
        in_specs=[pl.BlockSpec((6, _R1), lambda i: (0, i))],
        out_specs=pl.BlockSpec((_S, _R1), lambda i: (0, i)),
        out_shape=jax.ShapeDtypeStruct((_S, _N), jnp.int32),
    )(rays_t)


def _sc_gather_body(idx_hbm, sigma_hbm, out_hbm, idx_v, val_v, sem):
    wid = lax.axis_index("s") * _NC + lax.axis_index("c")
    base = wid * _PERW
    for ch in range(_NCH):
        off = base + ch * _K
        pltpu.sync_copy(idx_hbm.at[pl.ds(off, _K)], idx_v)
        pltpu.async_copy(sigma_hbm.at[idx_v], val_v, sem).wait()
        pltpu.sync_copy(val_v, out_hbm.at[pl.ds(off, _K)])


@jax.jit
def _gather_call(idx_flat, sigma_flat):
    k = pl.kernel(
        _sc_gather_body,
        out_type=jax.ShapeDtypeStruct((_TOT,), jnp.float32),
        mesh=plsc.VectorSubcoreMesh(core_axis_name="c", subcore_axis_name="s"),
        scratch_types=[
            pltpu.VMEM((_K,), jnp.int32),
            pltpu.VMEM((_K,), jnp.float32),
            pltpu.SemaphoreType.DMA,
        ],
    )
    return k(idx_flat, sigma_flat)


def _composite_body(vals_ref, rays_ref, tri_ref, out_ref):
    _, _, near, far, ns, ns_f = _ray_setup(rays_ref)
    dist = (far - near) / ns_f
    ji = lax.broadcasted_iota(jnp.int32, (_S, _R2), 0)
    mask = ji < ns
    s_u = jnp.maximum(vals_ref[...], 0.0) * dist
    s_m = jnp.where(mask, s_u, 0.0)
    s_cum = jnp.dot(tri_ref[...], s_m, preferred_element_type=jnp.float32)
    cumprod = jnp.exp(s_cum)
    alpha = 1.0 - jnp.exp(s_u)
    w1 = 1.0 + jnp.where(mask, alpha * cumprod, 0.0)
    h = _S // 2
    while h >= 1:
        w1 = w1[0:h, :] * w1[h:2 * h, :]
        h //= 2
    c = w1
    out_ref[...] = jnp.concatenate([c, c, c, 1.0 - c], axis=0)


@jax.jit
def _composite_call(vals_t, rays_t, tri):
    return pl.pallas_call(
        _composite_body,
        grid=(_N // _R2,),
        in_specs=[
            pl.BlockSpec((_S, _R2), lambda i: (0, i)),
            pl.BlockSpec((6, _R2), lambda i: (0, i)),
            pl.BlockSpec((_S, _S), lambda i: (0, 0)),
        ],
        out_specs=pl.BlockSpec((4, _R2), lambda i: (0, i)),
        out_shape=jax.ShapeDtypeStruct((4, _N), jnp.float32),
    )(vals_t, rays_t, tri)


def kernel(w_sigma, w_rgb, rays):
    del w_rgb
    rays_t = rays.T
    sigma_flat = w_sigma.reshape(-1)
    idx_t = _idx_call(rays_t)
    vals = _gather_call(idx_t.reshape(-1), sigma_flat)
    tri = jnp.asarray(np.tril(np.ones((_S, _S), np.float32)))
    out4 = _composite_call(vals.reshape(_S, _N), rays_t, tri)
    return out4.T


# speedup vs baseline: 108.4094x; 1.1269x over previous
import functools

import numpy as np
import jax
import jax.numpy as jnp
from jax import lax
from jax.experimental import pallas as pl
from jax.experimental.pallas import tpu as pltpu
from jax.experimental.pallas import tpu_sc as plsc

_GRID = 128
_S = 256
_N = 8192
_BIG = 1e30
_VOX = _GRID ** 3
_H = _VOX // 2

_NC = 2
_NS = 16
_TOT = _S * _N
_PERC = _TOT // _NS
_K = 8192
_NCH = _PERC // _K
_SEG = _H // _NS

_R1 = 512
_R2 = 512


def _ray_setup(rays_ref):
    ox = rays_ref[0:1, :]
    oy = rays_ref[1:2, :]
    oz = rays_ref[2:3, :]
    rx = rays_ref[3:4, :]
    ry = rays_ref[4:5, :]
    rz = rays_ref[5:6, :]
    norm = jnp.sqrt(rx * rx + ry * ry + rz * rz)
    nears = []
    fars = []
    oks = []
    ds = []
    os_ = (ox, oy, oz)
    for o, draw in zip(os_, (rx, ry, rz)):
        d = draw / norm
        ds.append(d)
        zero = d == 0.0
        safe_d = jnp.where(zero, 1.0, d)
        i1 = jnp.where(zero, -_BIG, (-1.5 - o) / safe_d)
        i2 = jnp.where(zero, _BIG, (1.5 - o) / safe_d)
        nears.append(jnp.minimum(i1, i2))
        fars.append(jnp.maximum(i1, i2))
        inside = (o >= -1.5) & (o <= 1.5)
        oks.append(jnp.logical_or(~zero, inside))
    near = jnp.maximum(jnp.maximum(nears[0], nears[1]), nears[2])
    far = jnp.minimum(jnp.minimum(fars[0], fars[1]), fars[2])
    ok = oks[0] & oks[1] & oks[2]
    isect = (near <= far) & ok
    ns = jnp.where(isect,
                   jnp.minimum((far - near) * 32.0, 256.0).astype(jnp.int32),
                   0)
    ns_f = jnp.maximum(ns, 1).astype(jnp.float32)
    return os_, ds, near, far, ns, ns_f


def _voxel_indices(os_, ds, near, far, ns_f, r):
    j = lax.broadcasted_iota(jnp.int32, (_S, r), 0).astype(jnp.float32)
    t = near + (far - near) * (j + 0.5) / ns_f
    ips = []
    for o, d in zip(os_, ds):
        pos = (o + d * t) / 1.5 * 0.5 + 0.5
        p = pos * float(_GRID)
        ips.append(jnp.clip(jnp.floor(p).astype(jnp.int32), 0, _GRID - 1))
    return ips


def _idx_body(rays_ref, idx0_ref, idx1_ref):
    os_, ds, near, far, ns, ns_f = _ray_setup(rays_ref)
    ips = _voxel_indices(os_, ds, near, far, ns_f, _R1)
    flat = (ips[0] * _GRID + ips[1]) * _GRID + ips[2]
    idx0_ref[...] = jnp.minimum(flat, _H - 1)
    idx1_ref[...] = jnp.maximum(flat - _H, 0)


@jax.jit
def _idx_call(rays_t):
    return pl.pallas_call(
        _idx_body,
        grid=(_N // _R1,),
        in_specs=[pl.BlockSpec((6, _R1), lambda i: (0, i))],
        out_specs=[pl.BlockSpec((_S, _R1), lambda i: (0, i)),
                   pl.BlockSpec((_S, _R1), lambda i: (0, i))],
        out_shape=[jax.ShapeDtypeStruct((_S, _N), jnp.int32),
                   jax.ShapeDtypeStruct((_S, _N), jnp.int32)],
    )(rays_t)


def _core_loop(idx_hbm, out_hbm, out_base, tab_sp, bufs, sems, sid):
    base = sid * _PERC
    semi, semg, semo = sems
    loads = {}
    stores = {}
    loads[0] = pltpu.async_copy(idx_hbm.at[pl.ds(base, _K)], bufs[0][0], semi)
    for ch in range(_NCH):
        cur = ch % 2
        nxt = 1 - cur
        idx_c, val_c = bufs[cur]
        if ch + 1 < _NCH:
            off_n = base + (ch + 1) * _K
            loads[ch + 1] = pltpu.async_copy(
                idx_hbm.at[pl.ds(off_n, _K)], bufs[nxt][0], semi)
        loads[ch].wait()
        if ch >= 2:
            stores[ch - 2].wait()
        pltpu.async_copy(tab_sp.at[idx_c], val_c, semg).wait()
        off = out_base + base + ch * _K
        stores[ch] = pltpu.async_copy(
            val_c, out_hbm.at[pl.ds(off, _K)], semo)
    stores[_NCH - 2].wait()
    stores[_NCH - 1].wait()


def _sc_gather_body(idx0_hbm, idx1_hbm, sigma_hbm, out_hbm,
                    tab_sp, idx_v0, idx_v1, val_v0, val_v1,
                    semi, semg, semo):
    cid = lax.axis_index("c")
    sid = lax.axis_index("s")
    pltpu.sync_copy(sigma_hbm.at[pl.ds(cid * _H + sid * _SEG, _SEG)],
                    tab_sp.at[pl.ds(sid * _SEG, _SEG)])
    plsc.subcore_barrier()
    sems = (semi, semg, semo)
    bufs = [(idx_v0, val_v0), (idx_v1, val_v1)]

    @pl.when(cid == 0)
    def _core0():
        _core_loop(idx0_hbm, out_hbm, 0, tab_sp, bufs, sems, sid)

    @pl.when(cid == 1)
    def _core1():
        _core_loop(idx1_hbm, out_hbm, _TOT, tab_sp, bufs, sems, sid)


@jax.jit
def _gather_call(idx0, idx1, sigma_flat):
    k = pl.kernel(
        _sc_gather_body,
        out_type=jax.ShapeDtypeStruct((2 * _TOT,), jnp.float32),
        mesh=plsc.VectorSubcoreMesh(core_axis_name="c", subcore_axis_name="s"),
        scratch_types=[
            pltpu.VMEM_SHARED((_H,), jnp.float32),
            pltpu.VMEM((_K,), jnp.int32),
            pltpu.VMEM((_K,), jnp.int32),
            pltpu.VMEM((_K,), jnp.float32),
            pltpu.VMEM((_K,), jnp.float32),
            pltpu.SemaphoreType.DMA,
            pltpu.SemaphoreType.DMA,
            pltpu.SemaphoreType.DMA,
        ],
    )
    return k(idx0, idx1, sigma_flat)


def _composite_body(vals0_ref, vals1_ref, rays_ref, tri_ref, out_ref):
    os_, ds, near, far, ns, ns_f = _ray_setup(rays_ref)
    dist = (far - near) / ns_f
    ji = lax.broadcasted_iota(jnp.int32, (_S, _R2), 0)
    mask = ji < ns
    ips = _voxel_indices(os_, ds, near, far, ns_f, _R2)
    flat = (ips[0] * _GRID + ips[1]) * _GRID + ips[2]
    vals = jnp.where(flat < _H, vals0_ref[...], vals1_ref[...])
    s_u = jnp.maximum(vals, 0.0) * dist
    s_m = jnp.where(mask, s_u, 0.0)
    s_cum = jnp.dot(tri_ref[...], s_m, preferred_element_type=jnp.float32)
    cumprod = jnp.exp(s_cum)
    alpha = 1.0 - jnp.exp(s_u)
    w1 = 1.0 + jnp.where(mask, alpha * cumprod, 0.0)
    h = _S // 2
    while h >= 1:
        w1 = w1[0:h, :] * w1[h:2 * h, :]
        h //= 2
    c = w1
    out_ref[...] = jnp.concatenate([c, c, c, 1.0 - c], axis=0)


@jax.jit
def _composite_call(vals0, vals1, rays_t, tri):
    return pl.pallas_call(
        _composite_body,
        grid=(_N // _R2,),
        in_specs=[
            pl.BlockSpec((_S, _R2), lambda i: (0, i)),
            pl.BlockSpec((_S, _R2), lambda i: (0, i)),
            pl.BlockSpec((6, _R2), lambda i: (0, i)),
            pl.BlockSpec((_S, _S), lambda i: (0, 0)),
        ],
        out_specs=pl.BlockSpec((4, _R2), lambda i: (0, i)),
        out_shape=jax.ShapeDtypeStruct((4, _N), jnp.float32),
    )(vals0, vals1, rays_t, tri)


def kernel(w_sigma, w_rgb, rays):
    del w_rgb
    rays_t = rays.T
    sigma_flat = w_sigma.reshape(-1)
    idx0, idx1 = _idx_call(rays_t)
    vals = _gather_call(idx0.reshape(-1), idx1.reshape(-1), sigma_flat)
    vals2 = vals.reshape(2, _S, _N)
    tri = jnp.asarray(np.tril(np.ones((_S, _S), np.float32)))
    out4 = _composite_call(vals2[0], vals2[1], rays_t, tri)
    return out4.T


# speedup vs baseline: 258.6692x; 2.3860x over previous
import functools

import numpy as np
import jax
import jax.numpy as jnp
from jax import lax
from jax.experimental import pallas as pl
from jax.experimental.pallas import tpu as pltpu
from jax.experimental.pallas import tpu_sc as plsc

_GRID = 128
_S = 256
_SW = 192
_N = 8192
_BIG = 1e30
_VOX = _GRID ** 3
_H = _VOX // 2

_NC = 2
_NS = 16
_TOTW = _N * _SW
_RPW = _N // _NS
_BLK = 128
_NBLK = _RPW // _BLK
_SEG = _H // _NS

_R1 = 512
_R2 = 512


def _ray_setup(rays_ref, r):
    ox = rays_ref[:, 0:1]
    oy = rays_ref[:, 1:2]
    oz = rays_ref[:, 2:3]
    rx = rays_ref[:, 3:4]
    ry = rays_ref[:, 4:5]
    rz = rays_ref[:, 5:6]
    norm = jnp.sqrt(rx * rx + ry * ry + rz * rz)
    nears = []
    fars = []
    oks = []
    ds = []
    os_ = (ox, oy, oz)
    for o, draw in zip(os_, (rx, ry, rz)):
        d = draw / norm
        ds.append(d)
        zero = d == 0.0
        safe_d = jnp.where(zero, 1.0, d)
        i1 = jnp.where(zero, -_BIG, (-1.5 - o) / safe_d)
        i2 = jnp.where(zero, _BIG, (1.5 - o) / safe_d)
        nears.append(jnp.minimum(i1, i2))
        fars.append(jnp.maximum(i1, i2))
        inside = (o >= -1.5) & (o <= 1.5)
        oks.append(jnp.logical_or(~zero, inside))
    near = jnp.maximum(jnp.maximum(nears[0], nears[1]), nears[2])
    far = jnp.minimum(jnp.minimum(fars[0], fars[1]), fars[2])
    ok = oks[0] & oks[1] & oks[2]
    isect = (near <= far) & ok
    ns = jnp.where(isect,
                   jnp.minimum((far - near) * 32.0, 256.0).astype(jnp.int32),
                   0)
    ns_f = jnp.maximum(ns, 1).astype(jnp.float32)
    return os_, ds, near, far, ns, ns_f


def _voxel_indices(os_, ds, near, far, ns_f, r, s):
    j = lax.broadcasted_iota(jnp.int32, (r, s), 1).astype(jnp.float32)
    t = near + (far - near) * (j + 0.5) / ns_f
    ips = []
    for o, d in zip(os_, ds):
        pos = (o + d * t) / 1.5 * 0.5 + 0.5
        p = pos * float(_GRID)
        ips.append(jnp.clip(jnp.floor(p).astype(jnp.int32), 0, _GRID - 1))
    return ips


def _run_meta(m, ji):
    jmin = jnp.min(jnp.where(m, ji, _S), axis=1, keepdims=True)
    jend = jnp.max(jnp.where(m, ji + 1, 0), axis=1, keepdims=True)
    a = jmin & ~31
    l = jnp.maximum(0, (jend - a + 31) & ~31)
    return a, l


def _idx_body(rays_ref, idx0_ref, idx1_ref, meta0_ref, meta1_ref):
    os_, ds, near, far, ns, ns_f = _ray_setup(rays_ref, _R1)
    ips = _voxel_indices(os_, ds, near, far, ns_f, _R1, _SW)
    flat = (ips[0] * _GRID + ips[1]) * _GRID + ips[2]
    idx0_ref[...] = jnp.minimum(flat, _H - 1)
    idx1_ref[...] = jnp.maximum(flat - _H, 0)
    ji = lax.broadcasted_iota(jnp.int32, (_R1, _SW), 1)
    live = ji < ns
    a0, l0 = _run_meta(live & (ips[0] < _GRID // 2), ji)
    a1, l1 = _run_meta(live & (ips[0] >= _GRID // 2), ji)
    meta0_ref[...] = jnp.concatenate([a0, l0], axis=1)
    meta1_ref[...] = jnp.concatenate([a1, l1], axis=1)


@jax.jit
def _idx_call(rays):
    return pl.pallas_call(
        _idx_body,
        grid=(_N // _R1,),
        in_specs=[pl.BlockSpec((_R1, 6), lambda i: (i, 0))],
        out_specs=[pl.BlockSpec((_R1, _SW), lambda i: (i, 0)),
                   pl.BlockSpec((_R1, _SW), lambda i: (i, 0)),
                   pl.BlockSpec((_R1, 2), lambda i: (i, 0)),
                   pl.BlockSpec((_R1, 2), lambda i: (i, 0))],
        out_shape=[jax.ShapeDtypeStruct((_N, _SW), jnp.int32),
                   jax.ShapeDtypeStruct((_N, _SW), jnp.int32),
                   jax.ShapeDtypeStruct((_N, 2), jnp.int32),
                   jax.ShapeDtypeStruct((_N, 2), jnp.int32)],
    )(rays)


def _core_loop(idx_hbm, out_hbm, out_base, tab_sp, sm, idx_v, val_v, sem, sid):
    for blk in range(_NBLK):
        rblk = sid * _RPW + blk * _BLK
        pos0 = rblk * _SW
        pltpu.sync_copy(idx_hbm.at[pl.ds(pos0, _BLK * _SW)], idx_v)

        def ray_body(r, carry, blk=blk):
            rl = blk * _BLK + r
            a = pl.multiple_of(sm[2 * rl], 32)
            l = sm[2 * rl + 1]
            n32 = lax.shift_right_logical(l, 5)

            def tile_body(i, c):
                off = pl.multiple_of(r * _SW + a + 32 * i, 32)
                pltpu.async_copy(tab_sp.at[idx_v.at[pl.ds(off, 32)]],
                                 val_v.at[pl.ds(off, 32)], sem).wait()
                return c

            lax.fori_loop(0, n32, tile_body, 0)
            return carry

        lax.fori_loop(0, _BLK, ray_body, 0)
        pltpu.sync_copy(val_v, out_hbm.at[pl.ds(out_base + pos0, _BLK * _SW)])


def _sc_gather_body(idx0_hbm, idx1_hbm, meta0_hbm, meta1_hbm, sigma_hbm,
                    out_hbm, tab_sp, meta_sp, sm, idx_v, val_v, sem):
    cid = lax.axis_index("c")
    sid = lax.axis_index("s")
    pltpu.sync_copy(sigma_hbm.at[pl.ds(cid * _H + sid * _SEG, _SEG)],
                    tab_sp.at[pl.ds(sid * _SEG, _SEG)])

    @pl.when((sid == 0) & (cid == 0))
    def _meta0():
        pltpu.sync_copy(meta0_hbm, meta_sp)

    @pl.when((sid == 0) & (cid == 1))
    def _meta1():
        pltpu.sync_copy(meta1_hbm, meta_sp)

    plsc.subcore_barrier()
    pltpu.sync_copy(meta_sp.at[pl.ds(sid * (2 * _RPW), 2 * _RPW)], sm)

    @pl.when(cid == 0)
    def _core0():
        _core_loop(idx0_hbm, out_hbm, 0, tab_sp, sm, idx_v, val_v, sem, sid)

    @pl.when(cid == 1)
    def _core1():
        _core_loop(idx1_hbm, out_hbm, _TOTW, tab_sp, sm, idx_v, val_v, sem,
                   sid)


@jax.jit
def _gather_call(idx0, idx1, meta0, meta1, sigma_flat):
    k = pl.kernel(
        _sc_gather_body,
        out_type=jax.ShapeDtypeStruct((2 * _TOTW,), jnp.float32),
        mesh=plsc.VectorSubcoreMesh(core_axis_name="c", subcore_axis_name="s"),
        scratch_types=[
            pltpu.VMEM_SHARED((_H,), jnp.float32),
            pltpu.VMEM_SHARED((2 * _N,), jnp.int32),
            pltpu.SMEM((2 * _RPW,), jnp.int32),
            pltpu.VMEM((_BLK * _SW,), jnp.int32),
            pltpu.VMEM((_BLK * _SW,), jnp.float32),
            pltpu.SemaphoreType.DMA,
        ],
    )
    return k(idx0, idx1, meta0, meta1, sigma_flat)


def _composite_body(vals0_ref, vals1_ref, rays_ref, tri_ref, out_ref):
    os_, ds, near, far, ns, ns_f = _ray_setup(rays_ref, _R2)
    dist = (far - near) / ns_f
    ji = lax.broadcasted_iota(jnp.int32, (_R2, _S), 1)
    mask = ji < ns
    ips = _voxel_indices(os_, ds, near, far, ns_f, _R2, _S)
    flat = (ips[0] * _GRID + ips[1]) * _GRID + ips[2]
    zpad = jnp.zeros((_R2, _S - _SW), jnp.float32)
    v0 = jnp.concatenate([vals0_ref[...], zpad], axis=1)
    v1 = jnp.concatenate([vals1_ref[...], zpad], axis=1)
    vals = jnp.where(flat < _H, v0, v1)
    s_u = jnp.maximum(vals, 0.0) * dist
    s_m = jnp.where(mask, s_u, 0.0)
    s_cum = jnp.dot(s_m, tri_ref[...], preferred_element_type=jnp.float32)
    cumprod = jnp.exp(s_cum)
    alpha = 1.0 - jnp.exp(s_u)
    w1 = 1.0 + jnp.where(mask, alpha * cumprod, 0.0)
    h = _S // 2
    while h >= 1:
        w1 = w1[:, 0:h] * w1[:, h:2 * h]
        h //= 2
    c = w1
    out_ref[...] = jnp.concatenate([c, c, c, 1.0 - c], axis=1)


@jax.jit
def _composite_call(vals0, vals1, rays, tri):
    return pl.pallas_call(
        _composite_body,
        grid=(_N // _R2,),
        in_specs=[
            pl.BlockSpec((_R2, _SW), lambda i: (i, 0)),
            pl.BlockSpec((_R2, _SW), lambda i: (i, 0)),
            pl.BlockSpec((_R2, 6), lambda i: (i, 0)),
            pl.BlockSpec((_S, _S), lambda i: (0, 0)),
        ],
        out_specs=pl.BlockSpec((_R2, 4), lambda i: (i, 0)),
        out_shape=jax.ShapeDtypeStruct((_N, 4), jnp.float32),
    )(vals0, vals1, rays, tri)


def kernel(w_sigma, w_rgb, rays):
    del w_rgb
    sigma_flat = w_sigma.reshape(-1)
    idx0, idx1, meta0, meta1 = _idx_call(rays)
    vals = _gather_call(idx0.reshape(-1), idx1.reshape(-1),
                        meta0.reshape(-1), meta1.reshape(-1), sigma_flat)
    v2 = vals.reshape(2, _N, _SW)
    tri = jnp.asarray(np.triu(np.ones((_S, _S), np.float32)))
    return _composite_call(v2[0], v2[1], rays, tri)


# speedup vs baseline: 317.4724x; 1.2273x over previous
import functools

import numpy as np
import jax
import jax.numpy as jnp
from jax import lax
from jax.experimental import pallas as pl
from jax.experimental.pallas import tpu as pltpu
from jax.experimental.pallas import tpu_sc as plsc

_GRID = 128
_S = 256
_SW = 192
_N = 8192
_BIG = 1e30
_VOX = _GRID ** 3
_H = _VOX // 2

_NC = 2
_NS = 16
_TOTW = _N * _SW
_RPW = _N // _NS
_BLK = 128
_NBLK = _RPW // _BLK
_SEG = _H // _NS

_R1 = 512
_R2 = 512


def _ray_setup(rays_ref, r):
    ox = rays_ref[:, 0:1]
    oy = rays_ref[:, 1:2]
    oz = rays_ref[:, 2:3]
    rx = rays_ref[:, 3:4]
    ry = rays_ref[:, 4:5]
    rz = rays_ref[:, 5:6]
    norm = jnp.sqrt(rx * rx + ry * ry + rz * rz)
    nears = []
    fars = []
    oks = []
    ds = []
    os_ = (ox, oy, oz)
    for o, draw in zip(os_, (rx, ry, rz)):
        d = draw / norm
        ds.append(d)
        zero = d == 0.0
        safe_d = jnp.where(zero, 1.0, d)
        i1 = jnp.where(zero, -_BIG, (-1.5 - o) / safe_d)
        i2 = jnp.where(zero, _BIG, (1.5 - o) / safe_d)
        nears.append(jnp.minimum(i1, i2))
        fars.append(jnp.maximum(i1, i2))
        inside = (o >= -1.5) & (o <= 1.5)
        oks.append(jnp.logical_or(~zero, inside))
    near = jnp.maximum(jnp.maximum(nears[0], nears[1]), nears[2])
    far = jnp.minimum(jnp.minimum(fars[0], fars[1]), fars[2])
    ok = oks[0] & oks[1] & oks[2]
    isect = (near <= far) & ok
    ns = jnp.where(isect,
                   jnp.minimum((far - near) * 32.0, 256.0).astype(jnp.int32),
                   0)
    ns_f = jnp.maximum(ns, 1).astype(jnp.float32)
    return os_, ds, near, far, ns, ns_f


def _voxel_indices(os_, ds, near, far, ns_f, r, s):
    j = lax.broadcasted_iota(jnp.int32, (r, s), 1).astype(jnp.float32)
    t = near + (far - near) * (j + 0.5) / ns_f
    ips = []
    for o, d in zip(os_, ds):
        pos = (o + d * t) / 1.5 * 0.5 + 0.5
        p = pos * float(_GRID)
        ips.append(jnp.clip(jnp.floor(p).astype(jnp.int32), 0, _GRID - 1))
    return ips


def _run_meta(m, ji):
    jmin = jnp.min(jnp.where(m, ji, _S), axis=1, keepdims=True)
    jend = jnp.max(jnp.where(m, ji + 1, 0), axis=1, keepdims=True)
    a = jmin & ~31
    l = jnp.maximum(0, (jend - a + 31) & ~31)
    return a, l


def _idx_body(rays_ref, idx0_ref, idx1_ref, meta0_ref, meta1_ref):
    os_, ds, near, far, ns, ns_f = _ray_setup(rays_ref, _R1)
    ips = _voxel_indices(os_, ds, near, far, ns_f, _R1, _SW)
    flat = (ips[0] * _GRID + ips[1]) * _GRID + ips[2]
    idx0_ref[...] = jnp.minimum(flat, _H - 1)
    idx1_ref[...] = jnp.maximum(flat - _H, 0)
    ji = lax.broadcasted_iota(jnp.int32, (_R1, _SW), 1)
    live = ji < ns
    a0, l0 = _run_meta(live & (ips[0] < _GRID // 2), ji)
    a1, l1 = _run_meta(live & (ips[0] >= _GRID // 2), ji)
    meta0_ref[...] = a0 * 1024 + l0
    meta1_ref[...] = a1 * 1024 + l1


@jax.jit
def _idx_call(rays):
    return pl.pallas_call(
        _idx_body,
        grid=(_N // _R1,),
        in_specs=[pl.BlockSpec((_R1, 6), lambda i: (i, 0))],
        out_specs=[pl.BlockSpec((_R1, _SW), lambda i: (i, 0)),
                   pl.BlockSpec((_R1, _SW), lambda i: (i, 0)),
                   pl.BlockSpec((_R1, 1), lambda i: (i, 0)),
                   pl.BlockSpec((_R1, 1), lambda i: (i, 0))],
        out_shape=[jax.ShapeDtypeStruct((_N, _SW), jnp.int32),
                   jax.ShapeDtypeStruct((_N, _SW), jnp.int32),
                   jax.ShapeDtypeStruct((_N, 1), jnp.int32),
                   jax.ShapeDtypeStruct((_N, 1), jnp.int32)],
    )(rays)


def _core_loop(idx_hbm, out_hbm, out_base, tab_sp, sm, idx_v, val_v, dr_v,
               sem, sid):
    def drain_one(i, c):
        pltpu.make_async_copy(idx_hbm.at[pl.ds(0, 32)], dr_v, sem).wait()
        return c

    for blk in range(_NBLK):
        rblk = sid * _RPW + blk * _BLK
        pos0 = rblk * _SW
        pltpu.sync_copy(idx_hbm.at[pl.ds(pos0, _BLK * _SW)], idx_v)

        def ray_body(r, prev_n32, blk=blk):
            m = sm[blk * _BLK + r]
            a = pl.multiple_of(lax.shift_right_logical(m, 10), 32)
            n32 = lax.shift_right_logical(m & 1023, 5)

            def tile_body(i, c):
                off = pl.multiple_of(r * _SW + a + 32 * i, 32)
                pltpu.async_copy(tab_sp.at[idx_v.at[pl.ds(off, 32)]],
                                 val_v.at[pl.ds(off, 32)], sem)
                return c

            lax.fori_loop(0, n32, tile_body, 0)
            lax.fori_loop(0, prev_n32, drain_one, 0)
            return n32

        last = lax.fori_loop(0, _BLK, ray_body, 0)
        lax.fori_loop(0, last, drain_one, 0)
        pltpu.sync_copy(val_v, out_hbm.at[pl.ds(out_base + pos0, _BLK * _SW)])


def _sc_gather_body(idx0_hbm, idx1_hbm, meta0_hbm, meta1_hbm, sigma_hbm,
                    out_hbm, tab_sp, meta_sp, sm, idx_v, val_v, dr_v, sem):
    cid = lax.axis_index("c")
    sid = lax.axis_index("s")
    pltpu.sync_copy(sigma_hbm.at[pl.ds(cid * _H + sid * _SEG, _SEG)],
                    tab_sp.at[pl.ds(sid * _SEG, _SEG)])

    @pl.when((sid == 0) & (cid == 0))
    def _meta0():
        pltpu.sync_copy(meta0_hbm, meta_sp)

    @pl.when((sid == 0) & (cid == 1))
    def _meta1():
        pltpu.sync_copy(meta1_hbm, meta_sp)

    plsc.subcore_barrier()
    pltpu.sync_copy(meta_sp.at[pl.ds(sid * _RPW, _RPW)], sm)

    @pl.when(cid == 0)
    def _core0():
        _core_loop(idx0_hbm, out_hbm, 0, tab_sp, sm, idx_v, val_v, dr_v, sem,
                   sid)

    @pl.when(cid == 1)
    def _core1():
        _core_loop(idx1_hbm, out_hbm, _TOTW, tab_sp, sm, idx_v, val_v, dr_v,
                   sem, sid)


@jax.jit
def _gather_call(idx0, idx1, meta0, meta1, sigma_flat):
    k = pl.kernel(
        _sc_gather_body,
        out_type=jax.ShapeDtypeStruct((2 * _TOTW,), jnp.float32),
        mesh=plsc.VectorSubcoreMesh(core_axis_name="c", subcore_axis_name="s"),
        scratch_types=[
            pltpu.VMEM_SHARED((_H,), jnp.float32),
            pltpu.VMEM_SHARED((_N,), jnp.int32),
            pltpu.SMEM((_RPW,), jnp.int32),
            pltpu.VMEM((_BLK * _SW,), jnp.int32),
            pltpu.VMEM((_BLK * _SW,), jnp.float32),
            pltpu.VMEM((32,), jnp.int32),
            pltpu.SemaphoreType.DMA,
        ],
    )
    return k(idx0, idx1, meta0, meta1, sigma_flat)


def _composite_body(vals0_ref, vals1_ref, rays_ref, tri_ref, out_ref):
    os_, ds, near, far, ns, ns_f = _ray_setup(rays_ref, _R2)
    dist = (far - near) / ns_f
    ji = lax.broadcasted_iota(jnp.int32, (_R2, _S), 1)
    mask = ji < ns
    ips = _voxel_indices(os_, ds, near, far, ns_f, _R2, _S)
    flat = (ips[0] * _GRID + ips[1]) * _GRID + ips[2]
    zpad = jnp.zeros((_R2, _S - _SW), jnp.float32)
    v0 = jnp.concatenate([vals0_ref[...], zpad], axis=1)
    v1 = jnp.concatenate([vals1_ref[...], zpad], axis=1)
    vals = jnp.where(flat < _H, v0, v1)
    s_u = jnp.maximum(vals, 0.0) * dist
    s_m = jnp.where(mask, s_u, 0.0)
    s_cum = jnp.dot(s_m, tri_ref[...], preferred_element_type=jnp.float32)
    cumprod = jnp.exp(s_cum)
    alpha = 1.0 - jnp.exp(s_u)
    w1 = 1.0 + jnp.where(mask, alpha * cumprod, 0.0)
    h = _S // 2
    while h >= 1:
        w1 = w1[:, 0:h] * w1[:, h:2 * h]
        h //= 2
    c = w1
    out_ref[...] = jnp.concatenate([c, c, c, 1.0 - c], axis=1)


@jax.jit
def _composite_call(vals0, vals1, rays, tri):
    return pl.pallas_call(
        _composite_body,
        grid=(_N // _R2,),
        in_specs=[
            pl.BlockSpec((_R2, _SW), lambda i: (i, 0)),
            pl.BlockSpec((_R2, _SW), lambda i: (i, 0)),
            pl.BlockSpec((_R2, 6), lambda i: (i, 0)),
            pl.BlockSpec((_S, _S), lambda i: (0, 0)),
        ],
        out_specs=pl.BlockSpec((_R2, 4), lambda i: (i, 0)),
        out_shape=jax.ShapeDtypeStruct((_N, 4), jnp.float32),
    )(vals0, vals1, rays, tri)


def kernel(w_sigma, w_rgb, rays):
    del w_rgb
    sigma_flat = w_sigma.reshape(-1)
    idx0, idx1, meta0, meta1 = _idx_call(rays)
    vals = _gather_call(idx0.reshape(-1), idx1.reshape(-1),
                        meta0.reshape(-1), meta1.reshape(-1), sigma_flat)
    v2 = vals.reshape(2, _N, _SW)
    tri = jnp.asarray(np.triu(np.ones((_S, _S), np.float32)))
    return _composite_call(v2[0], v2[1], rays, tri)


# speedup vs baseline: 364.2641x; 1.1474x over previous
import functools

import numpy as np
import jax
import jax.numpy as jnp
from jax import lax
from jax.experimental import pallas as pl
from jax.experimental.pallas import tpu as pltpu
from jax.experimental.pallas import tpu_sc as plsc

_GRID = 128
_S = 256
_SW = 192
_N = 8192
_BIG = 1e30
_VOX = _GRID ** 3
_H = _VOX // 2

_NC = 2
_NS = 16
_TOTW = _N * _SW
_RPW = _N // _NS
_BLK = 128
_NBLK = _RPW // _BLK
_SEG = _H // _NS

_R1 = 512
_R2 = 512


def _ray_setup(rays_ref, r):
    ox = rays_ref[:, 0:1]
    oy = rays_ref[:, 1:2]
    oz = rays_ref[:, 2:3]
    rx = rays_ref[:, 3:4]
    ry = rays_ref[:, 4:5]
    rz = rays_ref[:, 5:6]
    norm = jnp.sqrt(rx * rx + ry * ry + rz * rz)
    nears = []
    fars = []
    oks = []
    ds = []
    os_ = (ox, oy, oz)
    for o, draw in zip(os_, (rx, ry, rz)):
        d = draw / norm
        ds.append(d)
        zero = d == 0.0
        safe_d = jnp.where(zero, 1.0, d)
        i1 = jnp.where(zero, -_BIG, (-1.5 - o) / safe_d)
        i2 = jnp.where(zero, _BIG, (1.5 - o) / safe_d)
        nears.append(jnp.minimum(i1, i2))
        fars.append(jnp.maximum(i1, i2))
        inside = (o >= -1.5) & (o <= 1.5)
        oks.append(jnp.logical_or(~zero, inside))
    near = jnp.maximum(jnp.maximum(nears[0], nears[1]), nears[2])
    far = jnp.minimum(jnp.minimum(fars[0], fars[1]), fars[2])
    ok = oks[0] & oks[1] & oks[2]
    isect = (near <= far) & ok
    ns = jnp.where(isect,
                   jnp.minimum((far - near) * 32.0, 256.0).astype(jnp.int32),
                   0)
    ns_f = jnp.maximum(ns, 1).astype(jnp.float32)
    return os_, ds, near, far, ns, ns_f


def _voxel_indices(os_, ds, near, far, ns_f, r, s):
    j = lax.broadcasted_iota(jnp.int32, (r, s), 1).astype(jnp.float32)
    t = near + (far - near) * (j + 0.5) / ns_f
    ips = []
    for o, d in zip(os_, ds):
        pos = (o + d * t) / 1.5 * 0.5 + 0.5
        p = pos * float(_GRID)
        ips.append(jnp.clip(jnp.floor(p).astype(jnp.int32), 0, _GRID - 1))
    return ips


def _run_meta(m, ji):
    jmin = jnp.min(jnp.where(m, ji, _S), axis=1, keepdims=True)
    jend = jnp.max(jnp.where(m, ji + 1, 0), axis=1, keepdims=True)
    a = jmin & ~31
    l = jnp.maximum(0, (jend - a + 31) & ~31)
    return a, l


def _idx_body(rays_ref, idx0_ref, idx1_ref, meta0_ref, meta1_ref, sel_ref):
    os_, ds, near, far, ns, ns_f = _ray_setup(rays_ref, _R1)
    ips = _voxel_indices(os_, ds, near, far, ns_f, _R1, _SW)
    flat = (ips[0] * _GRID + ips[1]) * _GRID + ips[2]
    idx0_ref[...] = jnp.minimum(flat, _H - 1)
    idx1_ref[...] = jnp.maximum(flat - _H, 0)
    sel_ref[...] = (flat >= _H).astype(jnp.int8)
    ji = lax.broadcasted_iota(jnp.int32, (_R1, _SW), 1)
    live = ji < ns
    a0, l0 = _run_meta(live & (ips[0] < _GRID // 2), ji)
    a1, l1 = _run_meta(live & (ips[0] >= _GRID // 2), ji)
    meta0_ref[...] = a0 * 1024 + l0
    meta1_ref[...] = a1 * 1024 + l1


@jax.jit
def _idx_call(rays):
    return pl.pallas_call(
        _idx_body,
        grid=(_N // _R1,),
        in_specs=[pl.BlockSpec((_R1, 6), lambda i: (i, 0))],
        out_specs=[pl.BlockSpec((_R1, _SW), lambda i: (i, 0)),
                   pl.BlockSpec((_R1, _SW), lambda i: (i, 0)),
                   pl.BlockSpec((_R1, 1), lambda i: (i, 0)),
                   pl.BlockSpec((_R1, 1), lambda i: (i, 0)),
                   pl.BlockSpec((_R1, _SW), lambda i: (i, 0))],
        out_shape=[jax.ShapeDtypeStruct((_N, _SW), jnp.int32),
                   jax.ShapeDtypeStruct((_N, _SW), jnp.int32),
                   jax.ShapeDtypeStruct((_N, 1), jnp.int32),
                   jax.ShapeDtypeStruct((_N, 1), jnp.int32),
                   jax.ShapeDtypeStruct((_N, _SW), jnp.int8)],
    )(rays)


def _core_loop(idx_hbm, out_hbm, out_base, tab_sp, sm, idx_v, val_v, dr_v,
               sem, sid):
    def drain_one(i, c):
        pltpu.make_async_copy(idx_hbm.at[pl.ds(0, 32)], dr_v, sem).wait()
        return c

    for blk in range(_NBLK):
        rblk = sid * _RPW + blk * _BLK
        pos0 = rblk * _SW
        pltpu.sync_copy(idx_hbm.at[pl.ds(pos0, _BLK * _SW)], idx_v)

        def ray_body(r, prev_n32, blk=blk):
            m = sm[blk * _BLK + r]
            a = pl.multiple_of(lax.shift_right_logical(m, 10), 32)
            n32 = lax.shift_right_logical(m & 1023, 5)

            def tile_body(i, c):
                off = pl.multiple_of(r * _SW + a + 32 * i, 32)
                pltpu.async_copy(tab_sp.at[idx_v.at[pl.ds(off, 32)]],
                                 val_v.at[pl.ds(off, 32)], sem)
                return c

            lax.fori_loop(0, n32, tile_body, 0)
            lax.fori_loop(0, prev_n32, drain_one, 0)
            return n32

        last = lax.fori_loop(0, _BLK, ray_body, 0)
        lax.fori_loop(0, last, drain_one, 0)
        pltpu.sync_copy(val_v, out_hbm.at[pl.ds(out_base + pos0, _BLK * _SW)])


def _sc_gather_body(idx0_hbm, idx1_hbm, meta0_hbm, meta1_hbm, sigma_hbm,
                    out0_hbm, out1_hbm, tab_sp, meta_sp, sm, idx_v, val_v,
                    dr_v, sem):
    cid = lax.axis_index("c")
    sid = lax.axis_index("s")
    pltpu.sync_copy(sigma_hbm.at[pl.ds(cid * _H + sid * _SEG, _SEG)],
                    tab_sp.at[pl.ds(sid * _SEG, _SEG)])

    @pl.when((sid == 0) & (cid == 0))
    def _meta0():
        pltpu.sync_copy(meta0_hbm, meta_sp)

    @pl.when((sid == 0) & (cid == 1))
    def _meta1():
        pltpu.sync_copy(meta1_hbm, meta_sp)

    plsc.subcore_barrier()
    pltpu.sync_copy(meta_sp.at[pl.ds(sid * _RPW, _RPW)], sm)

    @pl.when(cid == 0)
    def _core0():
        _core_loop(idx0_hbm, out0_hbm, 0, tab_sp, sm, idx_v, val_v, dr_v,
                   sem, sid)

    @pl.when(cid == 1)
    def _core1():
        _core_loop(idx1_hbm, out1_hbm, 0, tab_sp, sm, idx_v, val_v, dr_v,
                   sem, sid)


@jax.jit
def _gather_call(idx0, idx1, meta0, meta1, sigma_flat):
    k = pl.kernel(
        _sc_gather_body,
        out_type=(jax.ShapeDtypeStruct((_TOTW,), jnp.float32),
                  jax.ShapeDtypeStruct((_TOTW,), jnp.float32)),
        mesh=plsc.VectorSubcoreMesh(core_axis_name="c", subcore_axis_name="s"),
        scratch_types=[
            pltpu.VMEM_SHARED((_H,), jnp.float32),
            pltpu.VMEM_SHARED((_N,), jnp.int32),
            pltpu.SMEM((_RPW,), jnp.int32),
            pltpu.VMEM((_BLK * _SW,), jnp.int32),
            pltpu.VMEM((_BLK * _SW,), jnp.float32),
            pltpu.VMEM((32,), jnp.int32),
            pltpu.SemaphoreType.DMA,
        ],
    )
    return k(idx0, idx1, meta0, meta1, sigma_flat)


def _composite_body(vals0_ref, vals1_ref, sel_ref, rays_ref, tri_ref,
                    out_ref):
    os_, ds, near, far, ns, ns_f = _ray_setup(rays_ref, _R2)
    dist = (far - near) / ns_f
    ji = lax.broadcasted_iota(jnp.int32, (_R2, _S), 1)
    mask = ji < ns
    zpad = jnp.zeros((_R2, _S - _SW), jnp.float32)
    v0 = jnp.concatenate([vals0_ref[...], zpad], axis=1)
    v1 = jnp.concatenate([vals1_ref[...], zpad], axis=1)
    spad = jnp.zeros((_R2, _S - _SW), jnp.int8)
    sel = jnp.concatenate([sel_ref[...], spad], axis=1)
    vals = jnp.where(sel == 0, v0, v1)
    s_u = jnp.maximum(vals, 0.0) * dist
    s_m = jnp.where(mask, s_u, 0.0)
    s_cum = jnp.dot(s_m, tri_ref[...], preferred_element_type=jnp.float32)
    cumprod = jnp.exp(s_cum)
    alpha = 1.0 - jnp.exp(s_u)
    w1 = 1.0 + jnp.where(mask, alpha * cumprod, 0.0)
    h = _S // 2
    while h >= 1:
        w1 = w1[:, 0:h] * w1[:, h:2 * h]
        h //= 2
    c = w1
    out_ref[...] = jnp.concatenate([c, c, c, 1.0 - c], axis=1)


@jax.jit
def _composite_call(vals0, vals1, sel, rays, tri):
    return pl.pallas_call(
        _composite_body,
        grid=(_N // _R2,),
        in_specs=[
            pl.BlockSpec((_R2, _SW), lambda i: (i, 0)),
            pl.BlockSpec((_R2, _SW), lambda i: (i, 0)),
            pl.BlockSpec((_R2, _SW), lambda i: (i, 0)),
            pl.BlockSpec((_R2, 6), lambda i: (i, 0)),
            pl.BlockSpec((_S, _S), lambda i: (0, 0)),
        ],
        out_specs=pl.BlockSpec((_R2, 4), lambda i: (i, 0)),
        out_shape=jax.ShapeDtypeStruct((_N, 4), jnp.float32),
    )(vals0, vals1, sel, rays, tri)


def kernel(w_sigma, w_rgb, rays):
    del w_rgb
    sigma_flat = w_sigma.reshape(-1)
    idx0, idx1, meta0, meta1, sel = _idx_call(rays)
    vals0, vals1 = _gather_call(idx0.reshape(-1), idx1.reshape(-1),
                                meta0.reshape(-1), meta1.reshape(-1),
                                sigma_flat)
    tri = jnp.asarray(np.triu(np.ones((_S, _S), np.float32)))
    return _composite_call(vals0.reshape(_N, _SW), vals1.reshape(_N, _SW),
                           sel, rays, tri)


# speedup vs baseline: 413.4630x; 1.1351x over previous
import functools

import numpy as np
import jax
import jax.numpy as jnp
from jax import lax
from jax.experimental import pallas as pl
from jax.experimental.pallas import tpu as pltpu
from jax.experimental.pallas import tpu_sc as plsc

_GRID = 128
_S = 256
_SW = 192
_N = 8192
_BIG = 1e30
_VOX = _GRID ** 3
_H = _VOX // 2

_NC = 2
_NS = 16
_TOTW = _N * _SW
_RPW = _N // _NS
_BLK = 64
_NBLK = _RPW // _BLK
_SEG = _H // _NS

_R1 = 512
_R2 = 512


def _ray_setup(rays_ref, r):
    ox = rays_ref[:, 0:1]
    oy = rays_ref[:, 1:2]
    oz = rays_ref[:, 2:3]
    rx = rays_ref[:, 3:4]
    ry = rays_ref[:, 4:5]
    rz = rays_ref[:, 5:6]
    norm = jnp.sqrt(rx * rx + ry * ry + rz * rz)
    nears = []
    fars = []
    oks = []
    ds = []
    os_ = (ox, oy, oz)
    for o, draw in zip(os_, (rx, ry, rz)):
        d = draw / norm
        ds.append(d)
        zero = d == 0.0
        safe_d = jnp.where(zero, 1.0, d)
        i1 = jnp.where(zero, -_BIG, (-1.5 - o) / safe_d)
        i2 = jnp.where(zero, _BIG, (1.5 - o) / safe_d)
        nears.append(jnp.minimum(i1, i2))
        fars.append(jnp.maximum(i1, i2))
        inside = (o >= -1.5) & (o <= 1.5)
        oks.append(jnp.logical_or(~zero, inside))
    near = jnp.maximum(jnp.maximum(nears[0], nears[1]), nears[2])
    far = jnp.minimum(jnp.minimum(fars[0], fars[1]), fars[2])
    ok = oks[0] & oks[1] & oks[2]
    isect = (near <= far) & ok
    ns = jnp.where(isect,
                   jnp.minimum((far - near) * 32.0, 256.0).astype(jnp.int32),
                   0)
    ns_f = jnp.maximum(ns, 1).astype(jnp.float32)
    return os_, ds, near, far, ns, ns_f


def _voxel_indices(os_, ds, near, far, ns_f, r, s):
    j = lax.broadcasted_iota(jnp.int32, (r, s), 1).astype(jnp.float32)
    t = near + (far - near) * (j + 0.5) / ns_f
    ips = []
    for o, d in zip(os_, ds):
        pos = (o + d * t) / 1.5 * 0.5 + 0.5
        p = pos * float(_GRID)
        ips.append(jnp.clip(jnp.floor(p).astype(jnp.int32), 0, _GRID - 1))
    return ips


def _run_meta(m, ji):
    jmin = jnp.min(jnp.where(m, ji, _S), axis=1, keepdims=True)
    jend = jnp.max(jnp.where(m, ji + 1, 0), axis=1, keepdims=True)
    a = jmin & ~31
    l = jnp.maximum(0, (jend - a + 31) & ~31)
    return a, l


def _idx_body(rays_ref, idx0_ref, idx1_ref, meta0_ref, meta1_ref, sel_ref,
              dist_ref, ns_ref):
    os_, ds, near, far, ns, ns_f = _ray_setup(rays_ref, _R1)
    ips = _voxel_indices(os_, ds, near, far, ns_f, _R1, _SW)
    flat = (ips[0] * _GRID + ips[1]) * _GRID + ips[2]
    idx0_ref[...] = jnp.minimum(flat, _H - 1)
    idx1_ref[...] = jnp.maximum(flat - _H, 0)
    sel_ref[...] = (flat >= _H).astype(jnp.int8)
    dist_ref[...] = (far - near) / ns_f
    ns_ref[...] = ns
    ji = lax.broadcasted_iota(jnp.int32, (_R1, _SW), 1)
    live = ji < ns
    a0, l0 = _run_meta(live & (ips[0] < _GRID // 2), ji)
    a1, l1 = _run_meta(live & (ips[0] >= _GRID // 2), ji)
    meta0_ref[...] = a0 * 1024 + l0
    meta1_ref[...] = a1 * 1024 + l1


@jax.jit
def _idx_call(rays):
    return pl.pallas_call(
        _idx_body,
        grid=(_N // _R1,),
        in_specs=[pl.BlockSpec((_R1, 6), lambda i: (i, 0))],
        out_specs=[pl.BlockSpec((_R1, _SW), lambda i: (i, 0)),
                   pl.BlockSpec((_R1, _SW), lambda i: (i, 0)),
                   pl.BlockSpec((_R1, 1), lambda i: (i, 0)),
                   pl.BlockSpec((_R1, 1), lambda i: (i, 0)),
                   pl.BlockSpec((_R1, _SW), lambda i: (i, 0)),
                   pl.BlockSpec((_R1, 1), lambda i: (i, 0)),
                   pl.BlockSpec((_R1, 1), lambda i: (i, 0))],
        out_shape=[jax.ShapeDtypeStruct((_N, _SW), jnp.int32),
                   jax.ShapeDtypeStruct((_N, _SW), jnp.int32),
                   jax.ShapeDtypeStruct((_N, 1), jnp.int32),
                   jax.ShapeDtypeStruct((_N, 1), jnp.int32),
                   jax.ShapeDtypeStruct((_N, _SW), jnp.int8),
                   jax.ShapeDtypeStruct((_N, 1), jnp.float32),
                   jax.ShapeDtypeStruct((_N, 1), jnp.int32)],
    )(rays)


def _core_loop(idx_hbm, out_hbm, tab_sp, sm, idx_vs, val_vs, dr_v,
               semg, seml, sems, sid):
    def drain_one(i, c):
        pltpu.make_async_copy(idx_hbm.at[pl.ds(0, 32)], dr_v, semg).wait()
        return c

    def pos(blk):
        return (sid * _RPW + blk * _BLK) * _SW

    loads = {}
    stores = {}
    loads[0] = pltpu.async_copy(idx_hbm.at[pl.ds(pos(0), _BLK * _SW)],
                                idx_vs[0], seml)
    for blk in range(_NBLK):
        cur = blk & 1
        idx_v, val_v = idx_vs[cur], val_vs[cur]
        if blk + 1 < _NBLK:
            loads[blk + 1] = pltpu.async_copy(
                idx_hbm.at[pl.ds(pos(blk + 1), _BLK * _SW)],
                idx_vs[1 - cur], seml)
        loads[blk].wait()
        if blk >= 2:
            stores[blk - 2].wait()

        def ray_body(r, prev_n32, blk=blk, idx_v=idx_v, val_v=val_v):
            m = sm[blk * _BLK + r]
            a = pl.multiple_of(lax.shift_right_logical(m, 10), 32)
            n32 = lax.shift_right_logical(m & 1023, 5)

            def tile_body(i, c):
                off = pl.multiple_of(r * _SW + a + 32 * i, 32)
                pltpu.async_copy(tab_sp.at[idx_v.at[pl.ds(off, 32)]],
                                 val_v.at[pl.ds(off, 32)], semg)
                return c

            lax.fori_loop(0, n32, tile_body, 0)
            lax.fori_loop(0, prev_n32, drain_one, 0)
            return n32

        last = lax.fori_loop(0, _BLK, ray_body, 0)
        lax.fori_loop(0, last, drain_one, 0)
        stores[blk] = pltpu.async_copy(
            val_v, out_hbm.at[pl.ds(pos(blk), _BLK * _SW)], sems)
    stores[_NBLK - 2].wait()
    stores[_NBLK - 1].wait()


def _sc_gather_body(idx0_hbm, idx1_hbm, meta0_hbm, meta1_hbm, sigma_hbm,
                    out0_hbm, out1_hbm, tab_sp, meta_sp, sm, idx_v0, idx_v1,
                    val_v0, val_v1, dr_v, semg, seml, sems):
    cid = lax.axis_index("c")
    sid = lax.axis_index("s")
    pltpu.sync_copy(sigma_hbm.at[pl.ds(cid * _H + sid * _SEG, _SEG)],
                    tab_sp.at[pl.ds(sid * _SEG, _SEG)])

    @pl.when((sid == 0) & (cid == 0))
    def _meta0():
        pltpu.sync_copy(meta0_hbm, meta_sp)

    @pl.when((sid == 0) & (cid == 1))
    def _meta1():
        pltpu.sync_copy(meta1_hbm, meta_sp)

    plsc.subcore_barrier()
    pltpu.sync_copy(meta_sp.at[pl.ds(sid * _RPW, _RPW)], sm)

    idx_vs = (idx_v0, idx_v1)
    val_vs = (val_v0, val_v1)

    @pl.when(cid == 0)
    def _core0():
        _core_loop(idx0_hbm, out0_hbm, tab_sp, sm, idx_vs, val_vs, dr_v,
                   semg, seml, sems, sid)

    @pl.when(cid == 1)
    def _core1():
        _core_loop(idx1_hbm, out1_hbm, tab_sp, sm, idx_vs, val_vs, dr_v,
                   semg, seml, sems, sid)


@jax.jit
def _gather_call(idx0, idx1, meta0, meta1, sigma_flat):
    k = pl.kernel(
        _sc_gather_body,
        out_type=(jax.ShapeDtypeStruct((_TOTW,), jnp.float32),
                  jax.ShapeDtypeStruct((_TOTW,), jnp.float32)),
        mesh=plsc.VectorSubcoreMesh(core_axis_name="c", subcore_axis_name="s"),
        scratch_types=[
            pltpu.VMEM_SHARED((_H,), jnp.float32),
            pltpu.VMEM_SHARED((_N,), jnp.int32),
            pltpu.SMEM((_RPW,), jnp.int32),
            pltpu.VMEM((_BLK * _SW,), jnp.int32),
            pltpu.VMEM((_BLK * _SW,), jnp.int32),
            pltpu.VMEM((_BLK * _SW,), jnp.float32),
            pltpu.VMEM((_BLK * _SW,), jnp.float32),
            pltpu.VMEM((32,), jnp.int32),
            pltpu.SemaphoreType.DMA,
            pltpu.SemaphoreType.DMA,
            pltpu.SemaphoreType.DMA,
        ],
    )
    return k(idx0, idx1, meta0, meta1, sigma_flat)


def _composite_body(vals0_ref, vals1_ref, sel_ref, dist_ref, ns_ref,
                    tri_ref, out_ref):
    dist = dist_ref[...]
    ns = ns_ref[...]
    ji = lax.broadcasted_iota(jnp.int32, (_R2, _S), 1)
    mask = ji < ns
    zpad = jnp.zeros((_R2, _S - _SW), jnp.float32)
    v0 = jnp.concatenate([vals0_ref[...], zpad], axis=1)
    v1 = jnp.concatenate([vals1_ref[...], zpad], axis=1)
    spad = jnp.zeros((_R2, _S - _SW), jnp.int8)
    sel = jnp.concatenate([sel_ref[...], spad], axis=1)
    vals = jnp.where(sel == 0, v0, v1)
    s_u = jnp.maximum(vals, 0.0) * dist
    s_m = jnp.where(mask, s_u, 0.0)
    s_cum = jnp.dot(s_m, tri_ref[...], preferred_element_type=jnp.float32)
    cumprod = jnp.exp(s_cum)
    alpha = 1.0 - jnp.exp(s_u)
    w1 = 1.0 + jnp.where(mask, alpha * cumprod, 0.0)
    h = _S // 2
    while h >= 1:
        w1 = w1[:, 0:h] * w1[:, h:2 * h]
        h //= 2
    c = w1
    out_ref[...] = jnp.concatenate([c, c, c, 1.0 - c], axis=1)


@jax.jit
def _composite_call(vals0, vals1, sel, dist, ns, tri):
    return pl.pallas_call(
        _composite_body,
        grid=(_N // _R2,),
        in_specs=[
            pl.BlockSpec((_R2, _SW), lambda i: (i, 0)),
            pl.BlockSpec((_R2, _SW), lambda i: (i, 0)),
            pl.BlockSpec((_R2, _SW), lambda i: (i, 0)),
            pl.BlockSpec((_R2, 1), lambda i: (i, 0)),
            pl.BlockSpec((_R2, 1), lambda i: (i, 0)),
            pl.BlockSpec((_S, _S), lambda i: (0, 0)),
        ],
        out_specs=pl.BlockSpec((_R2, 4), lambda i: (i, 0)),
        out_shape=jax.ShapeDtypeStruct((_N, 4), jnp.float32),
    )(vals0, vals1, sel, dist, ns, tri)


def kernel(w_sigma, w_rgb, rays):
    del w_rgb
    sigma_flat = w_sigma.reshape(-1)
    idx0, idx1, meta0, meta1, sel, dist, ns = _idx_call(rays)
    vals0, vals1 = _gather_call(idx0.reshape(-1), idx1.reshape(-1),
                                meta0.reshape(-1), meta1.reshape(-1),
                                sigma_flat)
    tri = jnp.asarray(np.triu(np.ones((_S, _S), np.float32)))
    return _composite_call(vals0.reshape(_N, _SW), vals1.reshape(_N, _SW),
                           sel, dist, ns, tri)


# speedup vs baseline: 510.7011x; 1.2352x over previous
import functools

import numpy as np
import jax
import jax.numpy as jnp
from jax import lax
from jax.experimental import pallas as pl
from jax.experimental.pallas import tpu as pltpu
from jax.experimental.pallas import tpu_sc as plsc

_GRID = 128
_S = 256
_SW = 192
_N = 8192
_BIG = 1e30
_VOX = _GRID ** 3
_H = _VOX // 2

_NC = 2
_NS = 16
_TOTW = _N * _SW
_RPW = _N // _NS
_BLK = 64
_NBLK = _RPW // _BLK
_SEG = _H // _NS

_R1 = 512
_R2 = 512


def _ray_setup(rays_ref, r):
    ox = rays_ref[:, 0:1]
    oy = rays_ref[:, 1:2]
    oz = rays_ref[:, 2:3]
    rx = rays_ref[:, 3:4]
    ry = rays_ref[:, 4:5]
    rz = rays_ref[:, 5:6]
    norm = jnp.sqrt(rx * rx + ry * ry + rz * rz)
    nears = []
    fars = []
    oks = []
    ds = []
    os_ = (ox, oy, oz)
    for o, draw in zip(os_, (rx, ry, rz)):
        d = draw / norm
        ds.append(d)
        zero = d == 0.0
        safe_d = jnp.where(zero, 1.0, d)
        i1 = jnp.where(zero, -_BIG, (-1.5 - o) / safe_d)
        i2 = jnp.where(zero, _BIG, (1.5 - o) / safe_d)
        nears.append(jnp.minimum(i1, i2))
        fars.append(jnp.maximum(i1, i2))
        inside = (o >= -1.5) & (o <= 1.5)
        oks.append(jnp.logical_or(~zero, inside))
    near = jnp.maximum(jnp.maximum(nears[0], nears[1]), nears[2])
    far = jnp.minimum(jnp.minimum(fars[0], fars[1]), fars[2])
    ok = oks[0] & oks[1] & oks[2]
    isect = (near <= far) & ok
    ns = jnp.where(isect,
                   jnp.minimum((far - near) * 32.0, 256.0).astype(jnp.int32),
                   0)
    ns_f = jnp.maximum(ns, 1).astype(jnp.float32)
    return os_, ds, near, far, ns, ns_f


def _voxel_indices(os_, ds, near, far, ns_f, r, s):
    j = lax.broadcasted_iota(jnp.int32, (r, s), 1).astype(jnp.float32)
    t = near + (far - near) * (j + 0.5) / ns_f
    ips = []
    for o, d in zip(os_, ds):
        pos = (o + d * t) / 1.5 * 0.5 + 0.5
        p = pos * float(_GRID)
        ips.append(jnp.clip(jnp.floor(p).astype(jnp.int32), 0, _GRID - 1))
    return ips


def _run_meta(m, ji):
    jmin = jnp.min(jnp.where(m, ji, _S), axis=1, keepdims=True)
    jend = jnp.max(jnp.where(m, ji + 1, 0), axis=1, keepdims=True)
    a = jmin & ~31
    l = jnp.maximum(0, (jend - a + 31) & ~31)
    return a, l


def _idx_body(rays_ref, idx0_ref, idx1_ref, meta0_ref, meta1_ref, sel_ref,
              dist_ref, ns_ref):
    os_, ds, near, far, ns, ns_f = _ray_setup(rays_ref, _R1)
    ips = _voxel_indices(os_, ds, near, far, ns_f, _R1, _SW)
    flat = (ips[0] * _GRID + ips[1]) * _GRID + ips[2]
    idx0_ref[...] = jnp.minimum(flat, _H - 1)
    idx1_ref[...] = jnp.maximum(flat - _H, 0)
    sel_ref[...] = (flat >= _H).astype(jnp.int8)
    dist_ref[...] = (far - near) / ns_f
    ns_ref[...] = ns
    ji = lax.broadcasted_iota(jnp.int32, (_R1, _SW), 1)
    live = ji < ns
    a0, l0 = _run_meta(live & (ips[0] < _GRID // 2), ji)
    a1, l1 = _run_meta(live & (ips[0] >= _GRID // 2), ji)
    meta0_ref[...] = a0 * 1024 + l0
    meta1_ref[...] = a1 * 1024 + l1


def _idx_call(rays):
    n = rays.shape[0]
    return pl.pallas_call(
        _idx_body,
        grid=(n // _R1,),
        in_specs=[pl.BlockSpec((_R1, 6), lambda i: (i, 0))],
        out_specs=[pl.BlockSpec((_R1, _SW), lambda i: (i, 0)),
                   pl.BlockSpec((_R1, _SW), lambda i: (i, 0)),
                   pl.BlockSpec((_R1, 1), lambda i: (i, 0)),
                   pl.BlockSpec((_R1, 1), lambda i: (i, 0)),
                   pl.BlockSpec((_R1, _SW), lambda i: (i, 0)),
                   pl.BlockSpec((_R1, 1), lambda i: (i, 0)),
                   pl.BlockSpec((_R1, 1), lambda i: (i, 0))],
        out_shape=[jax.ShapeDtypeStruct((n, _SW), jnp.int32),
                   jax.ShapeDtypeStruct((n, _SW), jnp.int32),
                   jax.ShapeDtypeStruct((n, 1), jnp.int32),
                   jax.ShapeDtypeStruct((n, 1), jnp.int32),
                   jax.ShapeDtypeStruct((n, _SW), jnp.int8),
                   jax.ShapeDtypeStruct((n, 1), jnp.float32),
                   jax.ShapeDtypeStruct((n, 1), jnp.int32)],
    )(rays)


def _core_loop(idx_hbm, out_hbm, tab_sp, sm, idx_vs, val_vs, dr_v,
               semg, seml, sems, sid, rpw):
    def drain_one(i, c):
        pltpu.make_async_copy(idx_hbm.at[pl.ds(0, 32)], dr_v, semg).wait()
        return c

    nblk = rpw // _BLK

    def pos(blk):
        return (sid * rpw + blk * _BLK) * _SW

    loads = {}
    stores = {}
    loads[0] = pltpu.async_copy(idx_hbm.at[pl.ds(pos(0), _BLK * _SW)],
                                idx_vs[0], seml)
    for blk in range(nblk):
        cur = blk & 1
        idx_v, val_v = idx_vs[cur], val_vs[cur]
        if blk + 1 < nblk:
            loads[blk + 1] = pltpu.async_copy(
                idx_hbm.at[pl.ds(pos(blk + 1), _BLK * _SW)],
                idx_vs[1 - cur], seml)
        loads[blk].wait()
        if blk >= 2:
            stores[blk - 2].wait()

        def ray_body(r, prev_n32, blk=blk, idx_v=idx_v, val_v=val_v):
            m = sm[blk * _BLK + r]
            a = pl.multiple_of(lax.shift_right_logical(m, 10), 32)
            n32 = lax.shift_right_logical(m & 1023, 5)

            def tile_body(i, c):
                off = pl.multiple_of(r * _SW + a + 32 * i, 32)
                pltpu.async_copy(tab_sp.at[idx_v.at[pl.ds(off, 32)]],
                                 val_v.at[pl.ds(off, 32)], semg)
                return c

            lax.fori_loop(0, n32, tile_body, 0)
            lax.fori_loop(0, prev_n32, drain_one, 0)
            return n32

        last = lax.fori_loop(0, _BLK, ray_body, 0)
        lax.fori_loop(0, last, drain_one, 0)
        stores[blk] = pltpu.async_copy(
            val_v, out_hbm.at[pl.ds(pos(blk), _BLK * _SW)], sems)
    stores[nblk - 2].wait()
    stores[nblk - 1].wait()


def _sc_gather_body(idx0_hbm, idx1_hbm, meta0_hbm, meta1_hbm, sigma_hbm,
                    out0_hbm, out1_hbm, tab_sp, meta_sp, sm, idx_v0, idx_v1,
                    val_v0, val_v1, dr_v, semg, seml, sems):
    cid = lax.axis_index("c")
    sid = lax.axis_index("s")
    pltpu.sync_copy(sigma_hbm.at[pl.ds(cid * _H + sid * _SEG, _SEG)],
                    tab_sp.at[pl.ds(sid * _SEG, _SEG)])

    @pl.when((sid == 0) & (cid == 0))
    def _meta0():
        pltpu.sync_copy(meta0_hbm, meta_sp)

    @pl.when((sid == 0) & (cid == 1))
    def _meta1():
        pltpu.sync_copy(meta1_hbm, meta_sp)

    rpw = meta0_hbm.shape[0] // _NS
    plsc.subcore_barrier()
    pltpu.sync_copy(meta_sp.at[pl.ds(sid * rpw, rpw)], sm)

    idx_vs = (idx_v0, idx_v1)
    val_vs = (val_v0, val_v1)

    @pl.when(cid == 0)
    def _core0():
        _core_loop(idx0_hbm, out0_hbm, tab_sp, sm, idx_vs, val_vs, dr_v,
                   semg, seml, sems, sid, rpw)

    @pl.when(cid == 1)
    def _core1():
        _core_loop(idx1_hbm, out1_hbm, tab_sp, sm, idx_vs, val_vs, dr_v,
                   semg, seml, sems, sid, rpw)


def _gather_call(idx0, idx1, meta0, meta1, sigma_flat):
    n = meta0.shape[0]
    totw = n * _SW
    k = pl.kernel(
        _sc_gather_body,
        out_type=(jax.ShapeDtypeStruct((totw,), jnp.float32),
                  jax.ShapeDtypeStruct((totw,), jnp.float32)),
        mesh=plsc.VectorSubcoreMesh(core_axis_name="c", subcore_axis_name="s"),
        scratch_types=[
            pltpu.VMEM_SHARED((_H,), jnp.float32),
            pltpu.VMEM_SHARED((n,), jnp.int32),
            pltpu.SMEM((n // _NS,), jnp.int32),
            pltpu.VMEM((_BLK * _SW,), jnp.int32),
            pltpu.VMEM((_BLK * _SW,), jnp.int32),
            pltpu.VMEM((_BLK * _SW,), jnp.float32),
            pltpu.VMEM((_BLK * _SW,), jnp.float32),
            pltpu.VMEM((32,), jnp.int32),
            pltpu.SemaphoreType.DMA,
            pltpu.SemaphoreType.DMA,
            pltpu.SemaphoreType.DMA,
        ],
    )
    return k(idx0, idx1, meta0, meta1, sigma_flat)


def _composite_body(vals0_ref, vals1_ref, sel_ref, dist_ref, ns_ref,
                    tri_ref, out_ref):
    dist = dist_ref[...]
    ns = ns_ref[...]
    ji = lax.broadcasted_iota(jnp.int32, (_R2, _S), 1)
    mask = ji < ns
    zpad = jnp.zeros((_R2, _S - _SW), jnp.float32)
    v0 = jnp.concatenate([vals0_ref[...], zpad], axis=1)
    v1 = jnp.concatenate([vals1_ref[...], zpad], axis=1)
    spad = jnp.zeros((_R2, _S - _SW), jnp.int8)
    sel = jnp.concatenate([sel_ref[...], spad], axis=1)
    vals = jnp.where(sel == 0, v0, v1)
    s_u = jnp.maximum(vals, 0.0) * dist
    s_m = jnp.where(mask, s_u, 0.0)
    s_cum = jnp.dot(s_m, tri_ref[...], preferred_element_type=jnp.float32)
    cumprod = jnp.exp(s_cum)
    alpha = 1.0 - jnp.exp(s_u)
    w1 = 1.0 + jnp.where(mask, alpha * cumprod, 0.0)
    h = _S // 2
    while h >= 1:
        w1 = w1[:, 0:h] * w1[:, h:2 * h]
        h //= 2
    c = w1
    out_ref[...] = jnp.concatenate([c, c, c, 1.0 - c], axis=1)


def _composite_call(vals0, vals1, sel, dist, ns, tri):
    n = sel.shape[0]
    return pl.pallas_call(
        _composite_body,
        grid=(n // _R2,),
        in_specs=[
            pl.BlockSpec((_R2, _SW), lambda i: (i, 0)),
            pl.BlockSpec((_R2, _SW), lambda i: (i, 0)),
            pl.BlockSpec((_R2, _SW), lambda i: (i, 0)),
            pl.BlockSpec((_R2, 1), lambda i: (i, 0)),
            pl.BlockSpec((_R2, 1), lambda i: (i, 0)),
            pl.BlockSpec((_S, _S), lambda i: (0, 0)),
        ],
        out_specs=pl.BlockSpec((_R2, 4), lambda i: (i, 0)),
        out_shape=jax.ShapeDtypeStruct((n, 4), jnp.float32),
    )(vals0, vals1, sel, dist, ns, tri)


def kernel(w_sigma, w_rgb, rays):
    del w_rgb
    sigma_flat = w_sigma.reshape(-1)
    tri = jnp.asarray(np.triu(np.ones((_S, _S), np.float32)))
    nh = _N // 2
    outs = []
    for h in range(2):
        r = lax.slice_in_dim(rays, h * nh, (h + 1) * nh)
        idx0, idx1, meta0, meta1, sel, dist, ns = _idx_call(r)
        vals0, vals1 = _gather_call(idx0.reshape(-1), idx1.reshape(-1),
                                    meta0.reshape(-1), meta1.reshape(-1),
                                    sigma_flat)
        outs.append(_composite_call(vals0.reshape(nh, _SW),
                                    vals1.reshape(nh, _SW),
                                    sel, dist, ns, tri))
    return jnp.concatenate(outs, axis=0)
